# baseline (speedup 1.0000x reference)
import jax
import jax.numpy as jnp
from jax import lax
from jax.experimental import pallas as pl
from jax.experimental.pallas import tpu as pltpu
from jax.experimental.pallas import tpu_sc as plsc

ENTITY_COUNT = 1000000
RELATION_COUNT = 1000
EMBED_DIM = 64
BATCH = 16384

NC = 2
NS = 16
L = 16
NW = NC * NS
ROWS_PER_W = BATCH // NW
CHUNK = 128
N_CHUNKS = ROWS_PER_W // CHUNK
PAIR = 2 * EMBED_DIM


def _complex_body(s_hbm, r_hbm, o_hbm, eim_hbm, rim_hbm, ere_hbm, rre_hbm,
                  out_hbm,
                  s_v, r_v, o_v, sp_v, op_v, rp_v,
                  sre_b, sim_b, ore_b, oim_b, rre_b, rim_b,
                  out_v, sem):
    wid = lax.axis_index("s") * NC + lax.axis_index("c")
    base = wid * ROWS_PER_W

    pltpu.sync_copy(s_hbm.at[pl.ds(base, ROWS_PER_W)], s_v)
    pltpu.sync_copy(r_hbm.at[pl.ds(base, ROWS_PER_W)], r_v)
    pltpu.sync_copy(o_hbm.at[pl.ds(base, ROWS_PER_W)], o_v)

    def halve(i, _):
        sl = pl.ds(i * L, L)
        sp_v[sl] = lax.shift_right_logical(s_v[sl], 1)
        op_v[sl] = lax.shift_right_logical(o_v[sl], 1)
        rp_v[sl] = lax.shift_right_logical(r_v[sl], 1)
        return 0

    lax.fori_loop(0, ROWS_PER_W // L, halve, 0)

    iota16 = lax.iota(jnp.int32, L)

    for ci in range(N_CHUNKS):
        sl = pl.ds(ci * CHUNK, CHUNK)
        copies = [
            pltpu.async_copy(ere_hbm.at[sp_v.at[sl]], sre_b, sem),
            pltpu.async_copy(eim_hbm.at[sp_v.at[sl]], sim_b, sem),
            pltpu.async_copy(ere_hbm.at[op_v.at[sl]], ore_b, sem),
            pltpu.async_copy(eim_hbm.at[op_v.at[sl]], oim_b, sem),
            pltpu.async_copy(rre_hbm.at[rp_v.at[sl]], rre_b, sem),
            pltpu.async_copy(rim_hbm.at[rp_v.at[sl]], rim_b, sem),
        ]
        for cp in copies:
            cp.wait()

        def group_body(g, _, ci=ci):
            rows = g * L + iota16
            soff = (s_v[pl.ds(ci * CHUNK + g * L, L)] & 1) * EMBED_DIM
            ooff = (o_v[pl.ds(ci * CHUNK + g * L, L)] & 1) * EMBED_DIM
            roff = (r_v[pl.ds(ci * CHUNK + g * L, L)] & 1) * EMBED_DIM

            def col_body(c, acc):
                sc = soff + c
                oc = ooff + c
                rc = roff + c
                sre = plsc.load_gather(sre_b, [rows, sc])
                sim = plsc.load_gather(sim_b, [rows, sc])
                ore = plsc.load_gather(ore_b, [rows, oc])
                oim = plsc.load_gather(oim_b, [rows, oc])
                rre = plsc.load_gather(rre_b, [rows, rc])
                rim = plsc.load_gather(rim_b, [rows, rc])
                return acc + ((sre * ore + sim * oim) * rre
                              + (sre * oim - sim * ore) * rim)

            acc = lax.fori_loop(0, EMBED_DIM, col_body,
                                jnp.zeros((L,), jnp.float32))
            out_v[pl.ds(ci * CHUNK + g * L, L)] = acc
            return 0

        lax.fori_loop(0, CHUNK // L, group_body, 0)

    pltpu.sync_copy(out_v, out_hbm.at[pl.ds(base, ROWS_PER_W)])


@jax.jit
def _complex_score(s, r, o, E_im, R_im, E_re, R_re):
    e_im2 = E_im.reshape(ENTITY_COUNT // 2, PAIR)
    r_im2 = R_im.reshape(RELATION_COUNT // 2, PAIR)
    e_re2 = E_re.reshape(ENTITY_COUNT // 2, PAIR)
    r_re2 = R_re.reshape(RELATION_COUNT // 2, PAIR)
    mesh = plsc.VectorSubcoreMesh(core_axis_name="c", subcore_axis_name="s",
                                  num_cores=NC, num_subcores=NS)
    kern = pl.kernel(
        _complex_body,
        out_type=jax.ShapeDtypeStruct((BATCH,), jnp.float32),
        mesh=mesh,
        scratch_types=[
            pltpu.VMEM((ROWS_PER_W,), jnp.int32),
            pltpu.VMEM((ROWS_PER_W,), jnp.int32),
            pltpu.VMEM((ROWS_PER_W,), jnp.int32),
            pltpu.VMEM((ROWS_PER_W,), jnp.int32),
            pltpu.VMEM((ROWS_PER_W,), jnp.int32),
            pltpu.VMEM((ROWS_PER_W,), jnp.int32),
            pltpu.VMEM((CHUNK, PAIR), jnp.float32),
            pltpu.VMEM((CHUNK, PAIR), jnp.float32),
            pltpu.VMEM((CHUNK, PAIR), jnp.float32),
            pltpu.VMEM((CHUNK, PAIR), jnp.float32),
            pltpu.VMEM((CHUNK, PAIR), jnp.float32),
            pltpu.VMEM((CHUNK, PAIR), jnp.float32),
            pltpu.VMEM((ROWS_PER_W,), jnp.float32),
            pltpu.SemaphoreType.DMA,
        ],
        compiler_params=pltpu.CompilerParams(needs_layout_passes=False),
    )
    return kern(s, r, o, e_im2, r_im2, e_re2, r_re2)


def kernel(s, r, o, E_im, R_im, E_re, R_re):
    s = s.astype(jnp.int32)
    r = r.astype(jnp.int32)
    o = o.astype(jnp.int32)
    return _complex_score(s, r, o, E_im, R_im, E_re, R_re)


# speedup vs baseline: 1.4880x; 1.4880x over previous
import jax
import jax.numpy as jnp
from jax import lax
from jax.experimental import pallas as pl
from jax.experimental.pallas import tpu as pltpu
from jax.experimental.pallas import tpu_sc as plsc
from jax.experimental.layout import Layout, with_layout_constraint

ENTITY_COUNT = 1000000
RELATION_COUNT = 1000
EMBED_DIM = 64
BATCH = 16384

NC = 2
NS = 16
L = 16
NW = NC * NS
ROWS_PER_W = BATCH // NW
WAVE = 64
N_WAVES = ROWS_PER_W // WAVE


def _complex_body(s_hbm, r_hbm, o_hbm, eim_hbm, rim_hbm, ere_hbm, rre_hbm,
                  out_hbm,
                  s_v, r_v, o_v,
                  sre_v, sim_v, ore_v, oim_v, rre_v, rim_v,
                  out_v, sem):
    wid = lax.axis_index("s") * NC + lax.axis_index("c")
    base = wid * ROWS_PER_W

    pltpu.sync_copy(s_hbm.at[pl.ds(base, ROWS_PER_W)], s_v)
    pltpu.sync_copy(r_hbm.at[pl.ds(base, ROWS_PER_W)], r_v)
    pltpu.sync_copy(o_hbm.at[pl.ds(base, ROWS_PER_W)], o_v)

    iota16 = lax.iota(jnp.int32, L)

    def wave_body(w, _):
        def issue(j, _):
            s16 = s_v[pl.ds(w * WAVE + j * L, L)]
            o16 = o_v[pl.ds(w * WAVE + j * L, L)]
            r16 = r_v[pl.ds(w * WAVE + j * L, L)]
            for k in range(L):
                i = j * L + k
                es = s16[k]
                eo = o16[k]
                er = r16[k]
                pltpu.make_async_copy(
                    ere_hbm.at[pl.ds(es, 1)], sre_v.at[pl.ds(i, 1)],
                    sem).start()
                pltpu.make_async_copy(
                    eim_hbm.at[pl.ds(es, 1)], sim_v.at[pl.ds(i, 1)],
                    sem).start()
                pltpu.make_async_copy(
                    ere_hbm.at[pl.ds(eo, 1)], ore_v.at[pl.ds(i, 1)],
                    sem).start()
                pltpu.make_async_copy(
                    eim_hbm.at[pl.ds(eo, 1)], oim_v.at[pl.ds(i, 1)],
                    sem).start()
                pltpu.make_async_copy(
                    rre_hbm.at[pl.ds(er, 1)], rre_v.at[pl.ds(i, 1)],
                    sem).start()
                pltpu.make_async_copy(
                    rim_hbm.at[pl.ds(er, 1)], rim_v.at[pl.ds(i, 1)],
                    sem).start()
            return 0

        lax.fori_loop(0, WAVE // L, issue, 0)

        def drain(i, _):
            pltpu.make_async_copy(
                ere_hbm.at[pl.ds(0, 1)], sre_v.at[pl.ds(i, 1)], sem).wait()
            pltpu.make_async_copy(
                eim_hbm.at[pl.ds(0, 1)], sim_v.at[pl.ds(i, 1)], sem).wait()
            pltpu.make_async_copy(
                ere_hbm.at[pl.ds(0, 1)], ore_v.at[pl.ds(i, 1)], sem).wait()
            pltpu.make_async_copy(
                eim_hbm.at[pl.ds(0, 1)], oim_v.at[pl.ds(i, 1)], sem).wait()
            pltpu.make_async_copy(
                rre_hbm.at[pl.ds(0, 1)], rre_v.at[pl.ds(i, 1)], sem).wait()
            pltpu.make_async_copy(
                rim_hbm.at[pl.ds(0, 1)], rim_v.at[pl.ds(i, 1)], sem).wait()
            return 0

        lax.fori_loop(0, WAVE, drain, 0)

        def group_body(g, _):
            rows = g * L + iota16

            def col_body(c, acc):
                cols = jnp.zeros((L,), jnp.int32) + c
                sre = plsc.load_gather(sre_v, [rows, cols])
                sim = plsc.load_gather(sim_v, [rows, cols])
                ore = plsc.load_gather(ore_v, [rows, cols])
                oim = plsc.load_gather(oim_v, [rows, cols])
                rre = plsc.load_gather(rre_v, [rows, cols])
                rim = plsc.load_gather(rim_v, [rows, cols])
                return acc + ((sre * ore + sim * oim) * rre
                              + (sre * oim - sim * ore) * rim)

            acc = lax.fori_loop(0, EMBED_DIM, col_body,
                                jnp.zeros((L,), jnp.float32))
            out_v[pl.ds(w * WAVE + g * L, L)] = acc
            return 0

        lax.fori_loop(0, WAVE // L, group_body, 0)
        return 0

    lax.fori_loop(0, N_WAVES, wave_body, 0)

    pltpu.sync_copy(out_v, out_hbm.at[pl.ds(base, ROWS_PER_W)])


@jax.jit
def _complex_score(s, r, o, E_im, R_im, E_re, R_re):
    fmt = Layout(major_to_minor=(1, 0), tiling=((8, 128),))
    E_im = with_layout_constraint(E_im, fmt)
    E_re = with_layout_constraint(E_re, fmt)
    R_im = with_layout_constraint(R_im, fmt)
    R_re = with_layout_constraint(R_re, fmt)
    mesh = plsc.VectorSubcoreMesh(core_axis_name="c", subcore_axis_name="s",
                                  num_cores=NC, num_subcores=NS)
    kern = pl.kernel(
        _complex_body,
        out_type=jax.ShapeDtypeStruct((BATCH,), jnp.float32),
        mesh=mesh,
        scratch_types=[
            pltpu.VMEM((ROWS_PER_W,), jnp.int32),
            pltpu.VMEM((ROWS_PER_W,), jnp.int32),
            pltpu.VMEM((ROWS_PER_W,), jnp.int32),
            pltpu.VMEM((WAVE, EMBED_DIM), jnp.float32),
            pltpu.VMEM((WAVE, EMBED_DIM), jnp.float32),
            pltpu.VMEM((WAVE, EMBED_DIM), jnp.float32),
            pltpu.VMEM((WAVE, EMBED_DIM), jnp.float32),
            pltpu.VMEM((WAVE, EMBED_DIM), jnp.float32),
            pltpu.VMEM((WAVE, EMBED_DIM), jnp.float32),
            pltpu.VMEM((ROWS_PER_W,), jnp.float32),
            pltpu.SemaphoreType.DMA,
        ],
        compiler_params=pltpu.CompilerParams(needs_layout_passes=False),
    )
    return kern(s, r, o, E_im, R_im, E_re, R_re)


def kernel(s, r, o, E_im, R_im, E_re, R_re):
    s = s.astype(jnp.int32)
    r = r.astype(jnp.int32)
    o = o.astype(jnp.int32)
    return _complex_score(s, r, o, E_im, R_im, E_re, R_re)


# speedup vs baseline: 1.4903x; 1.0016x over previous
import jax
import jax.numpy as jnp
from jax import lax
from jax.experimental import pallas as pl
from jax.experimental.pallas import tpu as pltpu
from jax.experimental.pallas import tpu_sc as plsc
from jax.experimental.layout import Layout, with_layout_constraint

ENTITY_COUNT = 1000000
RELATION_COUNT = 1000
EMBED_DIM = 64
BATCH = 16384

NC = 2
NS = 16
L = 16
NW = NC * NS
ROWS_PER_W = BATCH // NW
WAVE = 64
N_WAVES = ROWS_PER_W // WAVE


def _complex_body(s_hbm, r_hbm, o_hbm, eim_hbm, rim_hbm, ere_hbm, rre_hbm,
                  out_hbm,
                  s_v, r_v, o_v,
                  sre_v, sim_v, ore_v, oim_v, rre_v, rim_v,
                  out_v, sem):
    wid = lax.axis_index("s") * NC + lax.axis_index("c")
    base = wid * ROWS_PER_W

    pltpu.sync_copy(s_hbm.at[pl.ds(base, ROWS_PER_W)], s_v)
    pltpu.sync_copy(r_hbm.at[pl.ds(base, ROWS_PER_W)], r_v)
    pltpu.sync_copy(o_hbm.at[pl.ds(base, ROWS_PER_W)], o_v)

    iota16 = lax.iota(jnp.int32, L)

    def wave_body(w, _):
        def issue(j, _):
            s16 = s_v[pl.ds(w * WAVE + j * L, L)]
            o16 = o_v[pl.ds(w * WAVE + j * L, L)]
            r16 = r_v[pl.ds(w * WAVE + j * L, L)]
            for k in range(L):
                i = j * L + k
                es = s16[k]
                eo = o16[k]
                er = r16[k]
                pltpu.make_async_copy(
                    ere_hbm.at[pl.ds(es, 1)], sre_v.at[pl.ds(i, 1)],
                    sem).start()
                pltpu.make_async_copy(
                    eim_hbm.at[pl.ds(es, 1)], sim_v.at[pl.ds(i, 1)],
                    sem).start()
                pltpu.make_async_copy(
                    ere_hbm.at[pl.ds(eo, 1)], ore_v.at[pl.ds(i, 1)],
                    sem).start()
                pltpu.make_async_copy(
                    eim_hbm.at[pl.ds(eo, 1)], oim_v.at[pl.ds(i, 1)],
                    sem).start()
                pltpu.make_async_copy(
                    rre_hbm.at[pl.ds(er, 1)], rre_v.at[pl.ds(i, 1)],
                    sem).start()
                pltpu.make_async_copy(
                    rim_hbm.at[pl.ds(er, 1)], rim_v.at[pl.ds(i, 1)],
                    sem).start()
            return 0

        lax.fori_loop(0, WAVE // L, issue, 0)

        pltpu.make_async_copy(
            ere_hbm.at[pl.ds(0, WAVE)], sre_v, sem).wait()
        pltpu.make_async_copy(
            eim_hbm.at[pl.ds(0, WAVE)], sim_v, sem).wait()
        pltpu.make_async_copy(
            ere_hbm.at[pl.ds(0, WAVE)], ore_v, sem).wait()
        pltpu.make_async_copy(
            eim_hbm.at[pl.ds(0, WAVE)], oim_v, sem).wait()
        pltpu.make_async_copy(
            rre_hbm.at[pl.ds(0, WAVE)], rre_v, sem).wait()
        pltpu.make_async_copy(
            rim_hbm.at[pl.ds(0, WAVE)], rim_v, sem).wait()

        def group_body(g, _):
            rows = g * L + iota16

            def col_body(c, acc):
                cols = jnp.zeros((L,), jnp.int32) + c
                sre = plsc.load_gather(sre_v, [rows, cols])
                sim = plsc.load_gather(sim_v, [rows, cols])
                ore = plsc.load_gather(ore_v, [rows, cols])
                oim = plsc.load_gather(oim_v, [rows, cols])
                rre = plsc.load_gather(rre_v, [rows, cols])
                rim = plsc.load_gather(rim_v, [rows, cols])
                return acc + ((sre * ore + sim * oim) * rre
                              + (sre * oim - sim * ore) * rim)

            acc = lax.fori_loop(0, EMBED_DIM, col_body,
                                jnp.zeros((L,), jnp.float32))
            out_v[pl.ds(w * WAVE + g * L, L)] = acc
            return 0

        lax.fori_loop(0, WAVE // L, group_body, 0)
        return 0

    lax.fori_loop(0, N_WAVES, wave_body, 0)

    pltpu.sync_copy(out_v, out_hbm.at[pl.ds(base, ROWS_PER_W)])


@jax.jit
def _complex_score(s, r, o, E_im, R_im, E_re, R_re):
    fmt = Layout(major_to_minor=(1, 0), tiling=((8, 128),))
    E_im = with_layout_constraint(E_im, fmt)
    E_re = with_layout_constraint(E_re, fmt)
    R_im = with_layout_constraint(R_im, fmt)
    R_re = with_layout_constraint(R_re, fmt)
    mesh = plsc.VectorSubcoreMesh(core_axis_name="c", subcore_axis_name="s",
                                  num_cores=NC, num_subcores=NS)
    kern = pl.kernel(
        _complex_body,
        out_type=jax.ShapeDtypeStruct((BATCH,), jnp.float32),
        mesh=mesh,
        scratch_types=[
            pltpu.VMEM((ROWS_PER_W,), jnp.int32),
            pltpu.VMEM((ROWS_PER_W,), jnp.int32),
            pltpu.VMEM((ROWS_PER_W,), jnp.int32),
            pltpu.VMEM((WAVE, EMBED_DIM), jnp.float32),
            pltpu.VMEM((WAVE, EMBED_DIM), jnp.float32),
            pltpu.VMEM((WAVE, EMBED_DIM), jnp.float32),
            pltpu.VMEM((WAVE, EMBED_DIM), jnp.float32),
            pltpu.VMEM((WAVE, EMBED_DIM), jnp.float32),
            pltpu.VMEM((WAVE, EMBED_DIM), jnp.float32),
            pltpu.VMEM((ROWS_PER_W,), jnp.float32),
            pltpu.SemaphoreType.DMA,
        ],
        compiler_params=pltpu.CompilerParams(needs_layout_passes=False),
    )
    return kern(s, r, o, E_im, R_im, E_re, R_re)


def kernel(s, r, o, E_im, R_im, E_re, R_re):
    s = s.astype(jnp.int32)
    r = r.astype(jnp.int32)
    o = o.astype(jnp.int32)
    return _complex_score(s, r, o, E_im, R_im, E_re, R_re)
